# baseline (speedup 1.0000x reference)
import functools

import jax
import jax.numpy as jnp
from jax.experimental import pallas as pl
from jax.experimental.pallas import tpu as pltpu


def _round_up(x, m):
    return ((x + m - 1) // m) * m


def _lstm_kernel(xs0_ref, xsn_ref, h0_ref, c0_ref, wih_ref, whh_ref, b_ref,
                 hs_ref, h_out_ref, c_out_ref,
                 pre_ref, wih_b_ref, whh_b_ref,
                 *, tc, t_total, hidden, nc):
    n = pl.program_id(0)
    H = hidden
    B = xs0_ref.shape[1]
    I = xs0_ref.shape[2]

    @pl.when(n == 0)
    def _():
        h_out_ref[...] = h0_ref[...]
        c_out_ref[...] = c0_ref[...]
        wih_b_ref[...] = wih_ref[...].astype(jnp.bfloat16)
        whh_b_ref[...] = whh_ref[...].astype(jnp.bfloat16)
        x0 = xs0_ref[...].reshape(tc * B, I).astype(jnp.bfloat16)
        pre_ref[0] = (jnp.dot(x0, wih_b_ref[...],
                              preferred_element_type=jnp.float32)
                      + b_ref[...]).astype(jnp.bfloat16)

    xn = xsn_ref[...].reshape(tc * B, I).astype(jnp.bfloat16)
    nxt = jnp.dot(xn, wih_b_ref[...],
                  preferred_element_type=jnp.float32) + b_ref[...]
    pre_ref[(n + 1) % 2] = nxt.astype(jnp.bfloat16)

    cur = n % 2

    def step(s, carry):
        h, c = carry
        gates = pre_ref[cur, pl.ds(s * B, B), :].astype(jnp.float32) + jnp.dot(
            h.astype(jnp.bfloat16), whh_b_ref[...],
            preferred_element_type=jnp.float32)
        ifo = 0.5 * jnp.tanh(0.5 * gates[:, :3 * H]) + 0.5
        g = jnp.tanh(gates[:, 3 * H:])
        c_new = ifo[:, H:2 * H] * c + ifo[:, :H] * g
        h_new = ifo[:, 2 * H:3 * H] * jnp.tanh(c_new)
        if t_total % tc != 0:
            valid = (n * tc + s) < t_total
            h_new = jnp.where(valid, h_new, h)
            c_new = jnp.where(valid, c_new, c)
        hs_ref[s] = h_new
        return h_new, c_new

    h, c = jax.lax.fori_loop(0, tc, step, (h_out_ref[...], c_out_ref[...]),
                             unroll=True)
    h_out_ref[...] = h
    c_out_ref[...] = c


@functools.partial(jax.jit, static_argnames=("tc",))
def _fused_forward(xs, h0, c0, w_ih_t, w_hh_t, b, *, tc):
    T, B, I = xs.shape
    H = h0.shape[1]
    G4 = 4 * H

    Tp = _round_up(T, tc)
    if Tp != T:
        xs = jnp.pad(xs, ((0, Tp - T), (0, 0), (0, 0)))
    nc = Tp // tc

    b2 = b.reshape(1, G4)

    kernel_body = functools.partial(
        _lstm_kernel, tc=tc, t_total=T, hidden=H, nc=nc)

    out_shapes = (
        jax.ShapeDtypeStruct((Tp, B, H), jnp.float32),
        jax.ShapeDtypeStruct((B, H), jnp.float32),
        jax.ShapeDtypeStruct((B, H), jnp.float32),
    )

    last = nc - 1

    grid_spec = pltpu.PrefetchScalarGridSpec(
        num_scalar_prefetch=0,
        grid=(nc,),
        in_specs=[
            pl.BlockSpec((tc, B, I), lambda n: (0, 0, 0)),
            pl.BlockSpec((tc, B, I),
                         lambda n: (jnp.minimum(n + 1, last), 0, 0)),
            pl.BlockSpec((B, H), lambda n: (0, 0)),
            pl.BlockSpec((B, H), lambda n: (0, 0)),
            pl.BlockSpec((I, G4), lambda n: (0, 0)),
            pl.BlockSpec((H, G4), lambda n: (0, 0)),
            pl.BlockSpec((1, G4), lambda n: (0, 0)),
        ],
        out_specs=(
            pl.BlockSpec((tc, B, H), lambda n: (n, 0, 0)),
            pl.BlockSpec((B, H), lambda n: (0, 0)),
            pl.BlockSpec((B, H), lambda n: (0, 0)),
        ),
        scratch_shapes=[
            pltpu.VMEM((2, tc * B, G4), jnp.bfloat16),
            pltpu.VMEM((I, G4), jnp.bfloat16),
            pltpu.VMEM((H, G4), jnp.bfloat16),
        ],
    )

    hs, h, c = pl.pallas_call(
        kernel_body,
        out_shape=out_shapes,
        grid_spec=grid_spec,
        compiler_params=pltpu.CompilerParams(
            dimension_semantics=("arbitrary",)),
    )(xs, xs, h0, c0, w_ih_t, w_hh_t, b2)
    return hs[:T], h, c


def kernel(xs, h0, c0, w_ih_t, w_hh_t, b):
    return _fused_forward(xs, h0, c0, w_ih_t, w_hh_t, b, tc=32)


# speedup vs baseline: 1.0365x; 1.0365x over previous
import functools

import jax
import jax.numpy as jnp
from jax.experimental import pallas as pl
from jax.experimental.pallas import tpu as pltpu


def _round_up(x, m):
    return ((x + m - 1) // m) * m


def _lstm_kernel(xs0_ref, xsa_ref, xsb_ref, h0_ref, c0_ref,
                 wih_ref, whh_ref, b_ref,
                 hs_ref, h_out_ref, c_out_ref,
                 pre_a_ref, pre_b_ref, wih_b_ref, whh_b_ref,
                 *, tc, t_total, hidden, nc):
    m = pl.program_id(0)
    H = hidden
    B = xs0_ref.shape[1]
    I = xs0_ref.shape[2]
    bias = b_ref[...]

    def proj(x_ref, out_ref):
        x = x_ref[...].reshape(tc * B, I).astype(jnp.bfloat16)
        out_ref[...] = jnp.dot(x, wih_b_ref[...],
                               preferred_element_type=jnp.float32) + bias

    @pl.when(m == 0)
    def _():
        h_out_ref[...] = h0_ref[...]
        c_out_ref[...] = c0_ref[...]
        wih_b_ref[...] = wih_ref[...].astype(jnp.bfloat16)
        whh_b_ref[...] = whh_ref[...].astype(jnp.bfloat16)
        proj(xs0_ref, pre_a_ref)

    def run_chunk(pre_ref, hs_base, chunk, carry):
        def step(s, carry):
            h, c = carry
            gates = pre_ref[pl.ds(s * B, B), :] + jnp.dot(
                h.astype(jnp.bfloat16), whh_b_ref[...],
                preferred_element_type=jnp.float32)
            ifo = 0.5 * jnp.tanh(0.5 * gates[:, :3 * H]) + 0.5
            g = jnp.tanh(gates[:, 3 * H:])
            c_new = ifo[:, H:2 * H] * c + ifo[:, :H] * g
            h_new = ifo[:, 2 * H:3 * H] * jnp.tanh(c_new)
            if t_total % tc != 0:
                valid = (chunk * tc + s) < t_total
                h_new = jnp.where(valid, h_new, h)
                c_new = jnp.where(valid, c_new, c)
            hs_ref[hs_base + s] = h_new
            return h_new, c_new

        return jax.lax.fori_loop(0, tc, step, carry, unroll=True)

    carry = (h_out_ref[...], c_out_ref[...])
    proj(xsa_ref, pre_b_ref)
    carry = run_chunk(pre_a_ref, 0, 2 * m, carry)
    proj(xsb_ref, pre_a_ref)
    h, c = run_chunk(pre_b_ref, tc, 2 * m + 1, carry)
    h_out_ref[...] = h
    c_out_ref[...] = c


@functools.partial(jax.jit, static_argnames=("tc",))
def _fused_forward(xs, h0, c0, w_ih_t, w_hh_t, b, *, tc):
    T, B, I = xs.shape
    H = h0.shape[1]
    G4 = 4 * H

    Tp = _round_up(T, 2 * tc)
    if Tp != T:
        xs = jnp.pad(xs, ((0, Tp - T), (0, 0), (0, 0)))
    nc = Tp // tc
    nm = nc // 2

    b2 = b.reshape(1, G4)

    kernel_body = functools.partial(
        _lstm_kernel, tc=tc, t_total=T, hidden=H, nc=nc)

    out_shapes = (
        jax.ShapeDtypeStruct((Tp, B, H), jnp.float32),
        jax.ShapeDtypeStruct((B, H), jnp.float32),
        jax.ShapeDtypeStruct((B, H), jnp.float32),
    )

    last = nc - 1

    grid_spec = pltpu.PrefetchScalarGridSpec(
        num_scalar_prefetch=0,
        grid=(nm,),
        in_specs=[
            pl.BlockSpec((tc, B, I), lambda m: (0, 0, 0)),
            pl.BlockSpec((tc, B, I),
                         lambda m: (jnp.minimum(2 * m + 1, last), 0, 0)),
            pl.BlockSpec((tc, B, I),
                         lambda m: (jnp.minimum(2 * m + 2, last), 0, 0)),
            pl.BlockSpec((B, H), lambda m: (0, 0)),
            pl.BlockSpec((B, H), lambda m: (0, 0)),
            pl.BlockSpec((I, G4), lambda m: (0, 0)),
            pl.BlockSpec((H, G4), lambda m: (0, 0)),
            pl.BlockSpec((1, G4), lambda m: (0, 0)),
        ],
        out_specs=(
            pl.BlockSpec((2 * tc, B, H), lambda m: (m, 0, 0)),
            pl.BlockSpec((B, H), lambda m: (0, 0)),
            pl.BlockSpec((B, H), lambda m: (0, 0)),
        ),
        scratch_shapes=[
            pltpu.VMEM((tc * B, G4), jnp.float32),
            pltpu.VMEM((tc * B, G4), jnp.float32),
            pltpu.VMEM((I, G4), jnp.bfloat16),
            pltpu.VMEM((H, G4), jnp.bfloat16),
        ],
    )

    hs, h, c = pl.pallas_call(
        kernel_body,
        out_shape=out_shapes,
        grid_spec=grid_spec,
        compiler_params=pltpu.CompilerParams(
            dimension_semantics=("arbitrary",)),
    )(xs, xs, xs, h0, c0, w_ih_t, w_hh_t, b2)
    return hs[:T], h, c


def kernel(xs, h0, c0, w_ih_t, w_hh_t, b):
    return _fused_forward(xs, h0, c0, w_ih_t, w_hh_t, b, tc=16)


# speedup vs baseline: 1.0657x; 1.0282x over previous
import functools

import jax
import jax.numpy as jnp
from jax.experimental import pallas as pl
from jax.experimental.pallas import tpu as pltpu


def _round_up(x, m):
    return ((x + m - 1) // m) * m


def _lstm_kernel(xs_ref, h0_ref, c0_ref, wih_ref, whh_ref, b_ref,
                 hs_ref, h_out_ref, c_out_ref,
                 pre_ref, wih_b_ref, whh_b_ref,
                 *, tc, t_total, hidden):
    n = pl.program_id(0)
    H = hidden
    B = xs_ref.shape[1]

    @pl.when(n == 0)
    def _():
        h_out_ref[...] = h0_ref[...]
        c_out_ref[...] = c0_ref[...]
        wih_b_ref[...] = wih_ref[...].astype(jnp.bfloat16)
        whh_b_ref[...] = whh_ref[...].astype(jnp.bfloat16)

    x = xs_ref[...].reshape(tc * B, xs_ref.shape[2]).astype(jnp.bfloat16)
    pre_ref[...] = jnp.dot(x, wih_b_ref[...],
                           preferred_element_type=jnp.float32) + b_ref[...]

    def step(s, carry):
        h, c = carry
        gates = pre_ref[pl.ds(s * B, B), :] + jnp.dot(
            h.astype(jnp.bfloat16), whh_b_ref[...],
            preferred_element_type=jnp.float32)
        ifo = 0.5 * jnp.tanh(0.5 * gates[:, :3 * H]) + 0.5
        g = jnp.tanh(gates[:, 3 * H:])
        c_new = ifo[:, H:2 * H] * c + ifo[:, :H] * g
        h_new = ifo[:, 2 * H:3 * H] * jnp.tanh(c_new)
        if t_total % tc != 0:
            valid = (n * tc + s) < t_total
            h_new = jnp.where(valid, h_new, h)
            c_new = jnp.where(valid, c_new, c)
        hs_ref[s] = h_new
        return h_new, c_new

    h, c = jax.lax.fori_loop(0, tc, step, (h_out_ref[...], c_out_ref[...]),
                             unroll=True)
    h_out_ref[...] = h
    c_out_ref[...] = c


@functools.partial(jax.jit, static_argnames=("tc",))
def _fused_forward(xs, h0, c0, w_ih_t, w_hh_t, b, *, tc):
    T, B, I = xs.shape
    H = h0.shape[1]
    G4 = 4 * H

    Tp = _round_up(T, tc)
    if Tp != T:
        xs = jnp.pad(xs, ((0, Tp - T), (0, 0), (0, 0)))
    nc = Tp // tc

    b2 = b.reshape(1, G4)

    kernel_body = functools.partial(
        _lstm_kernel, tc=tc, t_total=T, hidden=H)

    out_shapes = (
        jax.ShapeDtypeStruct((Tp, B, H), jnp.float32),
        jax.ShapeDtypeStruct((B, H), jnp.float32),
        jax.ShapeDtypeStruct((B, H), jnp.float32),
    )

    grid_spec = pltpu.PrefetchScalarGridSpec(
        num_scalar_prefetch=0,
        grid=(nc,),
        in_specs=[
            pl.BlockSpec((tc, B, I), lambda n: (n, 0, 0)),
            pl.BlockSpec((B, H), lambda n: (0, 0)),
            pl.BlockSpec((B, H), lambda n: (0, 0)),
            pl.BlockSpec((I, G4), lambda n: (0, 0)),
            pl.BlockSpec((H, G4), lambda n: (0, 0)),
            pl.BlockSpec((1, G4), lambda n: (0, 0)),
        ],
        out_specs=(
            pl.BlockSpec((tc, B, H), lambda n: (n, 0, 0)),
            pl.BlockSpec((B, H), lambda n: (0, 0)),
            pl.BlockSpec((B, H), lambda n: (0, 0)),
        ),
        scratch_shapes=[
            pltpu.VMEM((tc * B, G4), jnp.float32),
            pltpu.VMEM((I, G4), jnp.bfloat16),
            pltpu.VMEM((H, G4), jnp.bfloat16),
        ],
    )

    hs, h, c = pl.pallas_call(
        kernel_body,
        out_shape=out_shapes,
        grid_spec=grid_spec,
        compiler_params=pltpu.CompilerParams(
            dimension_semantics=("arbitrary",)),
    )(xs, h0, c0, w_ih_t, w_hh_t, b2)
    return hs[:T], h, c


def kernel(xs, h0, c0, w_ih_t, w_hh_t, b):
    return _fused_forward(xs, h0, c0, w_ih_t, w_hh_t, b, tc=32)


# speedup vs baseline: 1.0780x; 1.0115x over previous
import functools

import jax
import jax.numpy as jnp
from jax.experimental import pallas as pl
from jax.experimental.pallas import tpu as pltpu


def _round_up(x, m):
    return ((x + m - 1) // m) * m


def _lstm_kernel(xs_ref, h0_ref, c0_ref, wih_ref, whh_ref, b_ref,
                 hs_ref, h_out_ref, c_out_ref,
                 pre_ref, wih_b_ref, whh_b_ref,
                 *, tc, t_total, hidden):
    n = pl.program_id(0)
    H = hidden
    B = xs_ref.shape[1]

    @pl.when(n == 0)
    def _():
        h_out_ref[...] = h0_ref[...]
        c_out_ref[...] = c0_ref[...]
        wih_b_ref[...] = wih_ref[...].astype(jnp.bfloat16)
        whh_b_ref[...] = whh_ref[...].astype(jnp.bfloat16)

    x = xs_ref[...].reshape(tc * B, xs_ref.shape[2]).astype(jnp.bfloat16)
    pre_ref[...] = jnp.dot(x, wih_b_ref[...],
                           preferred_element_type=jnp.float32) + b_ref[...]

    def step(s, carry):
        h, c = carry
        gates = pre_ref[pl.ds(s * B, B), :] + jnp.dot(
            h.astype(jnp.bfloat16), whh_b_ref[...],
            preferred_element_type=jnp.float32)
        ifo = 0.5 * jnp.tanh(0.5 * gates[:, :3 * H]) + 0.5
        g = jnp.tanh(gates[:, 3 * H:])
        c_new = ifo[:, H:2 * H] * c + ifo[:, :H] * g
        h_new = ifo[:, 2 * H:3 * H] * jnp.tanh(c_new)
        if t_total % tc != 0:
            valid = (n * tc + s) < t_total
            h_new = jnp.where(valid, h_new, h)
            c_new = jnp.where(valid, c_new, c)
        hs_ref[s] = h_new
        return h_new, c_new

    h, c = jax.lax.fori_loop(0, tc, step, (h_out_ref[...], c_out_ref[...]),
                             unroll=True)
    h_out_ref[...] = h
    c_out_ref[...] = c


@functools.partial(jax.jit, static_argnames=("tc",))
def _fused_forward(xs, h0, c0, w_ih_t, w_hh_t, b, *, tc):
    T, B, I = xs.shape
    H = h0.shape[1]
    G4 = 4 * H

    Tp = _round_up(T, tc)
    if Tp != T:
        xs = jnp.pad(xs, ((0, Tp - T), (0, 0), (0, 0)))
    nc = Tp // tc

    b2 = b.reshape(1, G4)

    kernel_body = functools.partial(
        _lstm_kernel, tc=tc, t_total=T, hidden=H)

    out_shapes = (
        jax.ShapeDtypeStruct((Tp, B, H), jnp.float32),
        jax.ShapeDtypeStruct((B, H), jnp.float32),
        jax.ShapeDtypeStruct((B, H), jnp.float32),
    )

    grid_spec = pltpu.PrefetchScalarGridSpec(
        num_scalar_prefetch=0,
        grid=(nc,),
        in_specs=[
            pl.BlockSpec((tc, B, I), lambda n: (n, 0, 0)),
            pl.BlockSpec((B, H), lambda n: (0, 0)),
            pl.BlockSpec((B, H), lambda n: (0, 0)),
            pl.BlockSpec((I, G4), lambda n: (0, 0)),
            pl.BlockSpec((H, G4), lambda n: (0, 0)),
            pl.BlockSpec((1, G4), lambda n: (0, 0)),
        ],
        out_specs=(
            pl.BlockSpec((tc, B, H), lambda n: (n, 0, 0)),
            pl.BlockSpec((B, H), lambda n: (0, 0)),
            pl.BlockSpec((B, H), lambda n: (0, 0)),
        ),
        scratch_shapes=[
            pltpu.VMEM((tc * B, G4), jnp.float32),
            pltpu.VMEM((I, G4), jnp.bfloat16),
            pltpu.VMEM((H, G4), jnp.bfloat16),
        ],
    )

    hs, h, c = pl.pallas_call(
        kernel_body,
        out_shape=out_shapes,
        grid_spec=grid_spec,
        compiler_params=pltpu.CompilerParams(
            dimension_semantics=("arbitrary",)),
    )(xs, h0, c0, w_ih_t, w_hh_t, b2)
    return hs[:T], h, c


def kernel(xs, h0, c0, w_ih_t, w_hh_t, b):
    return _fused_forward(xs, h0, c0, w_ih_t, w_hh_t, b, tc=16)
